# baseline (speedup 1.0000x reference)
import functools

import jax
import jax.numpy as jnp
from jax import lax
from jax.experimental import pallas as pl
from jax.experimental.pallas import tpu as pltpu
from jax.experimental.pallas import tpu_sc as plsc

EMBED_DIM = 64
EMB_SCALE = float(EMBED_DIM) ** 0.5
NUM_CORES = 2
NUM_SUBCORES = 16
NUM_WORKERS = NUM_CORES * NUM_SUBCORES
SPLITS = ((0, 104), (104, 96))
NBUF = 4
LANES = 16


def _scale_rows(rows_v, b, seq_len):
    def scale_body(r, _):
        for j in range(EMBED_DIM // LANES):
            sl = pl.ds(j * LANES, LANES)
            rows_v[b, r, sl] = rows_v[b, r, sl] * EMB_SCALE
        return ()

    lax.fori_loop(0, seq_len, scale_body, (), unroll=4)


def _emb_body(idx_hbm, table_hbm, out_hbm, idx_v, rows_v, gsems, ssems):
    rows_w, seq_len = idx_v.shape
    wid = lax.axis_index("s") * NUM_CORES + lax.axis_index("c")
    crow = wid * rows_w
    pltpu.sync_copy(idx_hbm.at[pl.ds(crow, rows_w)], idx_v)

    def start_gather(b, c):
        for off, sz in SPLITS:
            sl = pl.ds(off, sz)
            pltpu.async_copy(
                table_hbm.at[idx_v.at[c, sl]], rows_v.at[b, sl], gsems.at[b])

    def wait_gather(b, c):
        for off, sz in SPLITS:
            sl = pl.ds(off, sz)
            pltpu.make_async_copy(
                table_hbm.at[idx_v.at[c, sl]], rows_v.at[b, sl],
                gsems.at[b]).wait()

    def start_store(b, c):
        pltpu.async_copy(rows_v.at[b], out_hbm.at[crow + c], ssems.at[b])

    def wait_store(b, c):
        pltpu.make_async_copy(
            rows_v.at[b], out_hbm.at[crow + c], ssems.at[b]).wait()

    for b in range(NBUF):
        start_gather(b, b)

    n_groups = rows_w // NBUF

    def full_group(g, _):
        c0 = g * NBUF
        for b in range(NBUF):
            wait_gather(b, c0 + b)
            _scale_rows(rows_v, b, seq_len)
            start_store(b, c0 + b)
        for b in range(NBUF):
            wait_store(b, c0 + b)
            start_gather(b, c0 + b + NBUF)
        return ()

    lax.fori_loop(0, n_groups - 1, full_group, ())

    c0 = (n_groups - 1) * NBUF
    for b in range(NBUF):
        wait_gather(b, c0 + b)
        _scale_rows(rows_v, b, seq_len)
        start_store(b, c0 + b)
    for b in range(NBUF):
        wait_store(b, c0 + b)


def kernel(inputs, emb_weight):
    bsz, seq_len = inputs.shape
    assert bsz % NUM_WORKERS == 0 and seq_len == sum(s for _, s in SPLITS)
    rows_w = bsz // NUM_WORKERS
    assert rows_w % NBUF == 0

    mesh = plsc.VectorSubcoreMesh(core_axis_name="c", subcore_axis_name="s")
    emb = functools.partial(
        pl.kernel,
        mesh=mesh,
        out_type=jax.ShapeDtypeStruct((bsz, seq_len, EMBED_DIM), jnp.float32),
        compiler_params=pltpu.CompilerParams(use_tc_tiling_on_sc=False),
        scratch_types=[
            pltpu.VMEM((rows_w, seq_len), jnp.int32),
            pltpu.VMEM((NBUF, seq_len, EMBED_DIM), jnp.float32),
            pltpu.SemaphoreType.DMA((NBUF,)),
            pltpu.SemaphoreType.DMA((NBUF,)),
        ],
    )(_emb_body)
    return emb(inputs, emb_weight)


# speedup vs baseline: 1.1513x; 1.1513x over previous
import functools

import jax
import jax.numpy as jnp
from jax import lax
from jax.experimental import pallas as pl
from jax.experimental.pallas import tpu as pltpu
from jax.experimental.pallas import tpu_sc as plsc

EMBED_DIM = 64
ROW_PAD = 128
EMB_SCALE = float(EMBED_DIM) ** 0.5
NUM_CORES = 2
NUM_SUBCORES = 16
NUM_WORKERS = NUM_CORES * NUM_SUBCORES
SPLITS = ((0, 104), (104, 96))
NBUF = 4
LANES = 16


def _scale_rows(rows_v, b, seq_len):
    def scale_body(r, _):
        for j in range(EMBED_DIM // LANES):
            sl = pl.ds(j * LANES, LANES)
            rows_v[b, r, sl] = rows_v[b, r, sl] * EMB_SCALE
        return ()

    lax.fori_loop(0, seq_len, scale_body, (), unroll=4)


def _emb_body(idx_hbm, table_hbm, out_hbm, idx_v, rows_v, gsems, ssems):
    rows_w, seq_len = idx_v.shape
    wid = lax.axis_index("s") * NUM_CORES + lax.axis_index("c")
    crow = wid * rows_w
    pltpu.sync_copy(idx_hbm.at[pl.ds(crow, rows_w)], idx_v)

    def start_gather(b, c):
        for off, sz in SPLITS:
            pltpu.async_copy(
                table_hbm.at[idx_v.at[c, pl.ds(off, sz)]],
                rows_v.at[b, pl.ds(off, sz)], gsems.at[b])

    def wait_gather(b, c):
        for off, sz in SPLITS:
            pltpu.make_async_copy(
                table_hbm.at[idx_v.at[c, pl.ds(off, sz)]],
                rows_v.at[b, pl.ds(off, sz)], gsems.at[b]).wait()

    def start_store(b, c):
        pltpu.async_copy(
            rows_v.at[b, pl.ds(0, seq_len), pl.ds(0, EMBED_DIM)],
            out_hbm.at[crow + c, pl.ds(0, seq_len), pl.ds(0, EMBED_DIM)],
            ssems.at[b])

    def wait_store(b, c):
        pltpu.make_async_copy(
            rows_v.at[b, pl.ds(0, seq_len), pl.ds(0, EMBED_DIM)],
            out_hbm.at[crow + c, pl.ds(0, seq_len), pl.ds(0, EMBED_DIM)],
            ssems.at[b]).wait()

    for b in range(NBUF):
        start_gather(b, b)

    n_groups = rows_w // NBUF

    def full_group(g, _):
        c0 = g * NBUF
        for b in range(NBUF):
            wait_gather(b, c0 + b)
            _scale_rows(rows_v, b, seq_len)
            start_store(b, c0 + b)
        for b in range(NBUF):
            wait_store(b, c0 + b)
            start_gather(b, c0 + b + NBUF)
        return ()

    lax.fori_loop(0, n_groups - 1, full_group, ())

    c0 = (n_groups - 1) * NBUF
    for b in range(NBUF):
        wait_gather(b, c0 + b)
        _scale_rows(rows_v, b, seq_len)
        start_store(b, c0 + b)
    for b in range(NBUF):
        wait_store(b, c0 + b)


def kernel(inputs, emb_weight):
    bsz, seq_len = inputs.shape
    assert bsz % NUM_WORKERS == 0 and seq_len == sum(s for _, s in SPLITS)
    rows_w = bsz // NUM_WORKERS
    assert rows_w % NBUF == 0

    table_p = jnp.pad(emb_weight, ((0, 0), (0, ROW_PAD - EMBED_DIM)))

    mesh = plsc.VectorSubcoreMesh(core_axis_name="c", subcore_axis_name="s")
    emb = functools.partial(
        pl.kernel,
        mesh=mesh,
        out_type=jax.ShapeDtypeStruct((bsz, seq_len, ROW_PAD), jnp.float32),
        compiler_params=pltpu.CompilerParams(use_tc_tiling_on_sc=False),
        scratch_types=[
            pltpu.VMEM((rows_w, seq_len), jnp.int32),
            pltpu.VMEM((NBUF, seq_len, ROW_PAD), jnp.float32),
            pltpu.SemaphoreType.DMA((NBUF,)),
            pltpu.SemaphoreType.DMA((NBUF,)),
        ],
    )(_emb_body)
    out_p = emb(inputs, table_p)
    return out_p[:, :, :EMBED_DIM]


# speedup vs baseline: 1.3268x; 1.1524x over previous
import functools

import jax
import jax.numpy as jnp
from jax import lax
from jax.experimental import pallas as pl
from jax.experimental.pallas import tpu as pltpu
from jax.experimental.pallas import tpu_sc as plsc

EMBED_DIM = 64
ROW_PAD = 128
EMB_SCALE = float(EMBED_DIM) ** 0.5
NUM_CORES = 2
NUM_SUBCORES = 16
NUM_WORKERS = NUM_CORES * NUM_SUBCORES
SPLITS = ((0, 104), (104, 96))
NBUF = 4
LANES = 16


def _scale_rows(rows_v, b, seq_len):
    def scale_body(r, _):
        for j in range(EMBED_DIM // LANES):
            sl = pl.ds(j * LANES, LANES)
            rows_v[b, r, sl] = rows_v[b, r, sl] * EMB_SCALE
        return ()

    lax.fori_loop(0, seq_len, scale_body, (), unroll=4)


def _emb_body(idx_hbm, table_hbm, out_hbm, idx_v, rows_v, gsems, ssems):
    rows_w, seq_len = idx_v.shape
    wid = lax.axis_index("s") * NUM_CORES + lax.axis_index("c")
    crow = wid * rows_w
    pltpu.sync_copy(idx_hbm.at[pl.ds(crow, rows_w)], idx_v)

    def start_gather(b, c):
        for off, sz in SPLITS:
            pltpu.async_copy(
                table_hbm.at[idx_v.at[c, pl.ds(off, sz)]],
                rows_v.at[b, pl.ds(off, sz)], gsems.at[b])

    def wait_gather(b, c):
        for off, sz in SPLITS:
            pltpu.make_async_copy(
                table_hbm.at[idx_v.at[c, pl.ds(off, sz)]],
                rows_v.at[b, pl.ds(off, sz)], gsems.at[b]).wait()

    def start_store(b, c):
        pltpu.async_copy(
            rows_v.at[b, pl.ds(0, seq_len), pl.ds(0, EMBED_DIM)],
            out_hbm.at[crow + c, pl.ds(0, seq_len), pl.ds(0, EMBED_DIM)],
            ssems.at[b])

    def wait_store(b, c):
        pltpu.make_async_copy(
            rows_v.at[b, pl.ds(0, seq_len), pl.ds(0, EMBED_DIM)],
            out_hbm.at[crow + c, pl.ds(0, seq_len), pl.ds(0, EMBED_DIM)],
            ssems.at[b]).wait()

    for b in range(NBUF):
        start_gather(b, b)

    n_groups = rows_w // NBUF

    def full_group(g, _):
        c0 = g * NBUF
        for b in range(NBUF):
            wait_gather(b, c0 + b)
            _scale_rows(rows_v, b, seq_len)
            start_store(b, c0 + b)
        for b in range(NBUF):
            wait_store(b, c0 + b)
            start_gather(b, c0 + b + NBUF)
        return ()

    lax.fori_loop(0, n_groups - 1, full_group, ())

    c0 = (n_groups - 1) * NBUF
    for b in range(NBUF):
        wait_gather(b, c0 + b)
        _scale_rows(rows_v, b, seq_len)
        start_store(b, c0 + b)
    for b in range(NBUF):
        wait_store(b, c0 + b)


def kernel(inputs, emb_weight):
    bsz, seq_len = inputs.shape
    assert bsz % NUM_WORKERS == 0 and seq_len == sum(s for _, s in SPLITS)
    rows_w = bsz // NUM_WORKERS
    assert rows_w % NBUF == 0

    mesh = plsc.VectorSubcoreMesh(core_axis_name="c", subcore_axis_name="s")
    emb = functools.partial(
        pl.kernel,
        mesh=mesh,
        out_type=jax.ShapeDtypeStruct((bsz, seq_len, ROW_PAD), jnp.float32),
        compiler_params=pltpu.CompilerParams(use_tc_tiling_on_sc=False),
        scratch_types=[
            pltpu.VMEM((rows_w, seq_len), jnp.int32),
            pltpu.VMEM((NBUF, seq_len, EMBED_DIM), jnp.float32),
            pltpu.SemaphoreType.DMA((NBUF,)),
            pltpu.SemaphoreType.DMA((NBUF,)),
        ],
    )(_emb_body)
    out_p = emb(inputs, emb_weight)
    return out_p[:, :, :EMBED_DIM]


# speedup vs baseline: 1.3312x; 1.0034x over previous
import functools

import jax
import jax.numpy as jnp
from jax import lax
from jax.experimental import pallas as pl
from jax.experimental.pallas import tpu as pltpu
from jax.experimental.pallas import tpu_sc as plsc

EMBED_DIM = 64
ROW_PAD = 128
EMB_SCALE = float(EMBED_DIM) ** 0.5
NUM_CORES = 2
NUM_SUBCORES = 16
NUM_WORKERS = NUM_CORES * NUM_SUBCORES
SPLITS = ((0, 104), (104, 96))
NBUF = 8
LANES = 16


def _scale_rows(rows_v, b, seq_len):
    def scale_body(r, _):
        for j in range(EMBED_DIM // LANES):
            sl = pl.ds(j * LANES, LANES)
            rows_v[b, r, sl] = rows_v[b, r, sl] * EMB_SCALE
        return ()

    lax.fori_loop(0, seq_len, scale_body, (), unroll=4)


def _emb_body(idx_hbm, table_hbm, out_hbm, idx_v, rows_v, gsems, ssems):
    rows_w, seq_len = idx_v.shape
    wid = lax.axis_index("s") * NUM_CORES + lax.axis_index("c")
    crow = wid * rows_w
    pltpu.sync_copy(idx_hbm.at[pl.ds(crow, rows_w)], idx_v)

    def start_gather(b, c):
        for off, sz in SPLITS:
            pltpu.async_copy(
                table_hbm.at[idx_v.at[c, pl.ds(off, sz)]],
                rows_v.at[b, pl.ds(off, sz)], gsems.at[b])

    def wait_gather(b, c):
        for off, sz in SPLITS:
            pltpu.make_async_copy(
                table_hbm.at[idx_v.at[c, pl.ds(off, sz)]],
                rows_v.at[b, pl.ds(off, sz)], gsems.at[b]).wait()

    def start_store(b, c):
        pltpu.async_copy(
            rows_v.at[b, pl.ds(0, seq_len), pl.ds(0, EMBED_DIM)],
            out_hbm.at[crow + c, pl.ds(0, seq_len), pl.ds(0, EMBED_DIM)],
            ssems.at[b])

    def wait_store(b, c):
        pltpu.make_async_copy(
            rows_v.at[b, pl.ds(0, seq_len), pl.ds(0, EMBED_DIM)],
            out_hbm.at[crow + c, pl.ds(0, seq_len), pl.ds(0, EMBED_DIM)],
            ssems.at[b]).wait()

    for b in range(NBUF):
        start_gather(b, b)

    n_groups = rows_w // NBUF

    def full_group(g, _):
        c0 = g * NBUF
        for b in range(NBUF):
            wait_gather(b, c0 + b)
            _scale_rows(rows_v, b, seq_len)
            start_store(b, c0 + b)
        for b in range(NBUF):
            wait_store(b, c0 + b)
            start_gather(b, c0 + b + NBUF)
        return ()

    lax.fori_loop(0, n_groups - 1, full_group, ())

    c0 = (n_groups - 1) * NBUF
    for b in range(NBUF):
        wait_gather(b, c0 + b)
        _scale_rows(rows_v, b, seq_len)
        start_store(b, c0 + b)
    for b in range(NBUF):
        wait_store(b, c0 + b)


def kernel(inputs, emb_weight):
    bsz, seq_len = inputs.shape
    assert bsz % NUM_WORKERS == 0 and seq_len == sum(s for _, s in SPLITS)
    rows_w = bsz // NUM_WORKERS
    assert rows_w % NBUF == 0

    mesh = plsc.VectorSubcoreMesh(core_axis_name="c", subcore_axis_name="s")
    emb = functools.partial(
        pl.kernel,
        mesh=mesh,
        out_type=jax.ShapeDtypeStruct((bsz, seq_len, ROW_PAD), jnp.float32),
        compiler_params=pltpu.CompilerParams(use_tc_tiling_on_sc=False),
        scratch_types=[
            pltpu.VMEM((rows_w, seq_len), jnp.int32),
            pltpu.VMEM((NBUF, seq_len, EMBED_DIM), jnp.float32),
            pltpu.SemaphoreType.DMA((NBUF,)),
            pltpu.SemaphoreType.DMA((NBUF,)),
        ],
    )(_emb_body)
    out_p = emb(inputs, emb_weight)
    return out_p[:, :, :EMBED_DIM]
